# baseline (speedup 1.0000x reference)
import functools

import jax
import jax.numpy as jnp
from jax import lax
from jax.experimental import pallas as pl
from jax.experimental.pallas import tpu as pltpu
from jax.experimental.pallas import tpu_sc as plsc

NUM_CORES = 2
NUM_SUBCORES = 16
NUM_WORKERS = NUM_CORES * NUM_SUBCORES

CHUNK = 128
LANES = 16


def _sc_gather(tab128, idx_t, *, n_tok, seq, dim, pad_dim):
    toks_per_w = n_tok // NUM_WORKERS
    j_per_s = toks_per_w // CHUNK
    n_chunks = seq * j_per_s

    mesh = plsc.VectorSubcoreMesh(core_axis_name="c", subcore_axis_name="s")

    @functools.partial(
        pl.kernel,
        mesh=mesh,
        out_type=jax.ShapeDtypeStruct((seq * dim, n_tok), jnp.float32),
        compiler_params=pltpu.CompilerParams(use_tc_tiling_on_sc=True,
                                             needs_layout_passes=False),
        scratch_types=[
            pltpu.VMEM((seq, toks_per_w), jnp.int32),
            pltpu.VMEM((CHUNK, pad_dim), jnp.float32),
            pltpu.VMEM((CHUNK, pad_dim), jnp.float32),
            pltpu.VMEM((dim, CHUNK), jnp.float32),
            pltpu.VMEM((dim, CHUNK), jnp.float32),
            pltpu.SemaphoreType.DMA,
            pltpu.SemaphoreType.DMA,
            pltpu.SemaphoreType.DMA,
            pltpu.SemaphoreType.DMA,
        ],
    )
    def k(tab_hbm, idx_hbm, out_hbm, idx_v, buf_a, buf_b, tp_a, tp_b,
          gsem_a, gsem_b, ssem_a, ssem_b):
        wid = lax.axis_index("s") * NUM_CORES + lax.axis_index("c")
        tok_base = wid * toks_per_w

        pltpu.sync_copy(idx_hbm.at[:, pl.ds(tok_base, toks_per_w)], idx_v)

        def fire_gather(buf, gsem, c):
            s = c // j_per_s
            j = lax.rem(c, j_per_s)
            pltpu.async_copy(
                tab_hbm.at[idx_v.at[s, pl.ds(j * CHUNK, CHUNK)]],
                buf, gsem)

        def wait_gather(buf, gsem):
            pltpu.make_async_copy(tab_hbm.at[pl.ds(0, CHUNK)], buf,
                                  gsem).wait()

        def store_slice(c):
            s = c // j_per_s
            j = lax.rem(c, j_per_s)
            return out_hbm.at[pl.ds(s * dim, dim),
                              pl.ds(tok_base + j * CHUNK, CHUNK)]

        def transpose(buf, tp):
            iotas = [lax.iota(jnp.int32, LANES) + k16 * LANES
                     for k16 in range(CHUNK // LANES)]

            def col(f, _):
                fvec = jnp.full((LANES,), f, jnp.int32)
                for k16 in range(CHUNK // LANES):
                    vals = plsc.load_gather(buf, [iotas[k16], fvec])
                    tp[f, pl.ds(k16 * LANES, LANES)] = vals
                return _

            lax.fori_loop(0, dim, col, 0)

        def fire_store(tp, ssem, c):
            pltpu.async_copy(tp, store_slice(c), ssem)

        def wait_store(tp, ssem, c):
            pltpu.make_async_copy(tp, store_slice(c), ssem).wait()

        fire_gather(buf_a, gsem_a, 0)

        def step(c, buf, gbuf, gsem_cur, gsem_nxt, tp, ssem, *,
                 fire_next, wait_prev_store):
            wait_gather(buf, gsem_cur)
            if fire_next:
                fire_gather(gbuf, gsem_nxt, c + 1)
            if wait_prev_store:
                wait_store(tp, ssem, c - 2)
            transpose(buf, tp)
            fire_store(tp, ssem, c)

        step(0, buf_a, buf_b, gsem_a, gsem_b, tp_a, ssem_a,
             fire_next=True, wait_prev_store=False)
        step(1, buf_b, buf_a, gsem_b, gsem_a, tp_b, ssem_b,
             fire_next=True, wait_prev_store=False)

        def body(t, _):
            c = 2 * t
            step(c, buf_a, buf_b, gsem_a, gsem_b, tp_a, ssem_a,
                 fire_next=True, wait_prev_store=True)
            step(c + 1, buf_b, buf_a, gsem_b, gsem_a, tp_b, ssem_b,
                 fire_next=True, wait_prev_store=True)
            return _

        lax.fori_loop(1, n_chunks // 2 - 1, body, 0)

        step(n_chunks - 2, buf_a, buf_b, gsem_a, gsem_b, tp_a, ssem_a,
             fire_next=True, wait_prev_store=True)
        step(n_chunks - 1, buf_b, buf_a, gsem_b, gsem_a, tp_b, ssem_b,
             fire_next=False, wait_prev_store=True)
        wait_store(tp_a, ssem_a, n_chunks - 2)
        wait_store(tp_b, ssem_b, n_chunks - 1)

    return k(tab128, idx_t)


def kernel(token_ids, embeddings):
    n_tok, seq = token_ids.shape
    n_emb, dim = embeddings.shape
    pad_dim = 2 * dim
    tab128 = jnp.pad(embeddings, ((0, 0), (0, pad_dim - dim)))
    idx_t = token_ids.astype(jnp.int32).T
    out = _sc_gather(tab128, idx_t, n_tok=n_tok, seq=seq, dim=dim,
                     pad_dim=pad_dim)
    return out.reshape(seq, dim, n_tok).transpose(2, 0, 1)


# speedup vs baseline: 1.1961x; 1.1961x over previous
import functools

import jax
import jax.numpy as jnp
from jax import lax
from jax.experimental import pallas as pl
from jax.experimental.pallas import tpu as pltpu
from jax.experimental.pallas import tpu_sc as plsc

NUM_CORES = 2
NUM_SUBCORES = 16
NUM_WORKERS = NUM_CORES * NUM_SUBCORES

CHUNK = 128
LANES = 16


def _sc_gather(tab128, idx_t, *, n_tok, seq, dim, pad_dim):
    toks_per_w = n_tok // NUM_WORKERS
    j_per_s = toks_per_w // CHUNK
    n_chunks = seq * j_per_s

    mesh = plsc.VectorSubcoreMesh(core_axis_name="c", subcore_axis_name="s")

    @functools.partial(
        pl.kernel,
        mesh=mesh,
        out_type=jax.ShapeDtypeStruct((seq * dim, n_tok), jnp.float32),
        compiler_params=pltpu.CompilerParams(use_tc_tiling_on_sc=True,
                                             needs_layout_passes=False),
        scratch_types=[
            pltpu.VMEM((seq, toks_per_w), jnp.int32),
            pltpu.VMEM((CHUNK, pad_dim), jnp.float32),
            pltpu.VMEM((CHUNK, pad_dim), jnp.float32),
            pltpu.VMEM((dim, CHUNK), jnp.float32),
            pltpu.VMEM((dim, CHUNK), jnp.float32),
            pltpu.SemaphoreType.DMA,
            pltpu.SemaphoreType.DMA,
            pltpu.SemaphoreType.DMA,
            pltpu.SemaphoreType.DMA,
        ],
    )
    def k(tab_hbm, idx_hbm, out_hbm, idx_v, buf_a, buf_b, tp_a, tp_b,
          gsem_a, gsem_b, ssem_a, ssem_b):
        wid = lax.axis_index("s") * NUM_CORES + lax.axis_index("c")
        tok_base = wid * toks_per_w

        pltpu.sync_copy(idx_hbm.at[:, pl.ds(tok_base, toks_per_w)], idx_v)

        def fire_gather(buf, gsem, c):
            s = c // j_per_s
            j = lax.rem(c, j_per_s)
            pltpu.async_copy(
                tab_hbm.at[idx_v.at[s, pl.ds(j * CHUNK, CHUNK)]],
                buf, gsem)

        def wait_gather(buf, gsem):
            pltpu.make_async_copy(tab_hbm.at[pl.ds(0, CHUNK)], buf,
                                  gsem).wait()

        def store_slice(c):
            s = c // j_per_s
            j = lax.rem(c, j_per_s)
            return out_hbm.at[pl.ds(s * dim, dim),
                              pl.ds(tok_base + j * CHUNK, CHUNK)]

        def transpose(buf, tp):
            iotas = [lax.iota(jnp.int32, LANES) + k16 * LANES
                     for k16 in range(CHUNK // LANES)]
            nk = CHUNK // LANES
            cols_per_it = 4

            def colgrp(g, _):
                f0 = g * cols_per_it
                prev = None
                prev_f = None
                for cc in range(cols_per_it):
                    fvec = jnp.full((LANES,), f0 + cc, jnp.int32)
                    cur = []
                    for k16 in range(nk):
                        cur.append(plsc.load_gather(buf, [iotas[k16], fvec]))
                        if prev is not None:
                            tp[prev_f, pl.ds(k16 * LANES, LANES)] = prev[k16]
                    prev, prev_f = cur, f0 + cc
                for k16 in range(nk):
                    tp[prev_f, pl.ds(k16 * LANES, LANES)] = prev[k16]
                return _

            lax.fori_loop(0, dim // cols_per_it, colgrp, 0)

        def fire_store(tp, ssem, c):
            pltpu.async_copy(tp, store_slice(c), ssem)

        def wait_store(tp, ssem, c):
            pltpu.make_async_copy(tp, store_slice(c), ssem).wait()

        fire_gather(buf_a, gsem_a, 0)

        def step(c, buf, gbuf, gsem_cur, gsem_nxt, tp, ssem, *,
                 fire_next, wait_prev_store):
            wait_gather(buf, gsem_cur)
            if fire_next:
                fire_gather(gbuf, gsem_nxt, c + 1)
            if wait_prev_store:
                wait_store(tp, ssem, c - 2)
            transpose(buf, tp)
            fire_store(tp, ssem, c)

        step(0, buf_a, buf_b, gsem_a, gsem_b, tp_a, ssem_a,
             fire_next=True, wait_prev_store=False)
        step(1, buf_b, buf_a, gsem_b, gsem_a, tp_b, ssem_b,
             fire_next=True, wait_prev_store=False)

        def body(t, _):
            c = 2 * t
            step(c, buf_a, buf_b, gsem_a, gsem_b, tp_a, ssem_a,
                 fire_next=True, wait_prev_store=True)
            step(c + 1, buf_b, buf_a, gsem_b, gsem_a, tp_b, ssem_b,
                 fire_next=True, wait_prev_store=True)
            return _

        lax.fori_loop(1, n_chunks // 2 - 1, body, 0)

        step(n_chunks - 2, buf_a, buf_b, gsem_a, gsem_b, tp_a, ssem_a,
             fire_next=True, wait_prev_store=True)
        step(n_chunks - 1, buf_b, buf_a, gsem_b, gsem_a, tp_b, ssem_b,
             fire_next=False, wait_prev_store=True)
        wait_store(tp_a, ssem_a, n_chunks - 2)
        wait_store(tp_b, ssem_b, n_chunks - 1)

    return k(tab128, idx_t)


def kernel(token_ids, embeddings):
    n_tok, seq = token_ids.shape
    n_emb, dim = embeddings.shape
    pad_dim = 2 * dim
    tab128 = jnp.pad(embeddings, ((0, 0), (0, pad_dim - dim)))
    idx_t = token_ids.astype(jnp.int32).T
    out = _sc_gather(tab128, idx_t, n_tok=n_tok, seq=seq, dim=dim,
                     pad_dim=pad_dim)
    return out.reshape(seq, dim, n_tok).transpose(2, 0, 1)


# speedup vs baseline: 1.7449x; 1.4587x over previous
import functools

import jax
import jax.numpy as jnp
from jax import lax
from jax.experimental import pallas as pl
from jax.experimental.pallas import tpu as pltpu
from jax.experimental.pallas import tpu_sc as plsc

NUM_CORES = 2
NUM_SUBCORES = 16
NUM_WORKERS = NUM_CORES * NUM_SUBCORES

CHUNK = 128


def _sc_gather(tab128, idx_t, *, n_tok, seq, dim, row_w):
    toks_per_w = n_tok // NUM_WORKERS
    j_per_s = toks_per_w // CHUNK
    n_chunks = seq * j_per_s
    n_body = n_chunks // 2 - 1

    mesh = plsc.VectorSubcoreMesh(core_axis_name="c", subcore_axis_name="s")

    @functools.partial(
        pl.kernel,
        mesh=mesh,
        out_type=jax.ShapeDtypeStruct((n_tok, seq, row_w), jnp.float32),
        compiler_params=pltpu.CompilerParams(use_tc_tiling_on_sc=True,
                                             needs_layout_passes=False),
        scratch_types=[
            pltpu.VMEM((seq, toks_per_w), jnp.int32),
            pltpu.VMEM((CHUNK, 1, row_w), jnp.float32),
            pltpu.VMEM((CHUNK, 1, row_w), jnp.float32),
            pltpu.SemaphoreType.DMA,
            pltpu.SemaphoreType.DMA,
            pltpu.SemaphoreType.DMA,
            pltpu.SemaphoreType.DMA,
        ],
    )
    def k(tab_hbm, idx_hbm, out_hbm, idx_v, buf_a, buf_b,
          gsem_a, gsem_b, ssem_a, ssem_b):
        wid = lax.axis_index("s") * NUM_CORES + lax.axis_index("c")
        tok_base = wid * toks_per_w

        pltpu.sync_copy(idx_hbm.at[:, pl.ds(tok_base, toks_per_w)], idx_v)

        def fire_gather(buf, gsem, c):
            s = c // j_per_s
            j = lax.rem(c, j_per_s)
            pltpu.async_copy(
                tab_hbm.at[idx_v.at[s, pl.ds(j * CHUNK, CHUNK)]],
                buf.at[:, 0, :], gsem)

        def wait_gather(buf, gsem):
            pltpu.make_async_copy(tab_hbm.at[pl.ds(0, CHUNK)],
                                  buf.at[:, 0, :], gsem).wait()

        def store_pair(buf, c):
            s = c // j_per_s
            j = lax.rem(c, j_per_s)
            src = buf
            dst = out_hbm.at[pl.ds(tok_base + j * CHUNK, CHUNK),
                             pl.ds(s, 1), :]
            return src, dst

        def fire_store(buf, ssem, c):
            src, dst = store_pair(buf, c)
            pltpu.async_copy(src, dst, ssem)

        def wait_store(buf, ssem, c):
            src, dst = store_pair(buf, c)
            pltpu.make_async_copy(src, dst, ssem).wait()

        fire_gather(buf_a, gsem_a, 0)
        fire_gather(buf_b, gsem_b, 1)

        def body(t, _):
            c = 2 * t
            wait_gather(buf_a, gsem_a)
            fire_store(buf_a, ssem_a, c)
            wait_store(buf_a, ssem_a, c)
            fire_gather(buf_a, gsem_a, c + 2)
            wait_gather(buf_b, gsem_b)
            fire_store(buf_b, ssem_b, c + 1)
            wait_store(buf_b, ssem_b, c + 1)
            fire_gather(buf_b, gsem_b, c + 3)
            return _

        lax.fori_loop(0, n_body, body, 0)

        c_last = n_chunks - 2
        wait_gather(buf_a, gsem_a)
        fire_store(buf_a, ssem_a, c_last)
        wait_store(buf_a, ssem_a, c_last)
        wait_gather(buf_b, gsem_b)
        fire_store(buf_b, ssem_b, c_last + 1)
        wait_store(buf_b, ssem_b, c_last + 1)

    return k(tab128, idx_t)


def kernel(token_ids, embeddings):
    n_tok, seq = token_ids.shape
    n_emb, dim = embeddings.shape
    row_w = 2 * dim
    tab128 = jnp.pad(embeddings, ((0, 0), (0, row_w - dim)))
    idx_t = token_ids.astype(jnp.int32).T
    out = _sc_gather(tab128, idx_t, n_tok=n_tok, seq=seq, dim=dim,
                     row_w=row_w)
    return out[:, :, :dim]
